# baseline (speedup 1.0000x reference)
import jax
import jax.numpy as jnp
from jax import lax
from jax.experimental import pallas as pl
from jax.experimental.pallas import tpu as pltpu
from jax.experimental.pallas import tpu_sc as plsc

HOT_N, COLD_N, D, B = 1000, 10000, 256, 4096
OUT_N = HOT_N + B + COLD_N
NW = 32
L = 16
HOT_W = 32
COLD_W = 313
BS = (79, 78, 78, 78)
NBUF = 3
ACL_W = 320
B_PER_W = B // NW


def _zero_row(buf, i):
    z = jnp.zeros((L,), jnp.float32)
    for col in range(D // L):
        buf[i, pl.ds(col * L, L)] = z


def _body(hot_hbm, cold_hbm, ac_hbm, idx_hbm, out_hbm,
          idx_v, acl_v, acg_v, hotbuf, rows_v, flag_v, mrow_v,
          cbufs, sem_in, sem_hot, sem_g, sem_r2, lsems, ssems):
    wid = lax.axis_index("s") * 2 + lax.axis_index("c")
    lo = jnp.minimum(wid * COLD_W, COLD_N - COLD_W)
    alo = (lo // 8) * 8
    off = lo - alo

    starts = [sum(BS[:b]) for b in range(len(BS))]
    loads = [None] * len(BS)
    for b in range(NBUF):
        loads[b] = pltpu.async_copy(
            cold_hbm.at[pl.ds(lo + starts[b], BS[b])],
            cbufs[b].at[pl.ds(0, BS[b])], lsems[b])
    cp_idx = pltpu.async_copy(idx_hbm, idx_v, sem_in)
    cp_acl = pltpu.async_copy(ac_hbm.at[pl.ds(alo, ACL_W)],
                              acl_v.at[pl.ds(0, ACL_W)], sem_in)
    hstart = jnp.minimum(wid * HOT_W, HOT_N - HOT_W)
    cp_hl = pltpu.async_copy(hot_hbm.at[pl.ds(hstart, HOT_W)], hotbuf, sem_hot)

    cp_idx.wait()

    base2 = wid * B_PER_W
    idx2 = idx_v.at[pl.ds(base2, B_PER_W)]
    cp_g = pltpu.async_copy(cold_hbm.at[idx2], rows_v, sem_g)
    cp_ag = pltpu.async_copy(ac_hbm.at[idx2], acg_v, sem_g)

    cp_hl.wait()
    cp_hs = pltpu.async_copy(hotbuf, out_hbm.at[pl.ds(hstart, HOT_W)], sem_hot)

    def zbody(j, c):
        flag_v[pl.ds(j * L, L)] = jnp.zeros((L,), jnp.int32)
        return c

    lax.fori_loop(0, (COLD_W + 2 * L - 1) // L, zbody, 0)

    ones_i = jnp.ones((L,), jnp.int32)

    def fbody(j, c):
        for u in range(2):
            idx16 = idx_v[pl.ds((2 * j + u) * L, L)]
            m = (idx16 >= lo) & (idx16 < lo + COLD_W)
            loc = jnp.where(m, idx16 - lo, 0)
            plsc.store_scatter(flag_v, [loc], ones_i, mask=m)
        return c

    lax.fori_loop(0, B // (2 * L), fbody, 0)

    cp_acl.wait()

    def combody(j, c):
        hit = flag_v[pl.ds(j * L, L)]
        acb = acl_v[pl.ds(off + j * L, L)]
        keep = (hit > 0) & (acb > 5.0)
        flag_v[pl.ds(j * L, L)] = jnp.where(keep, 1, 0).astype(jnp.int32)
        return c

    lax.fori_loop(0, (COLD_W + L - 1) // L, combody, 0)

    cp_ag.wait()

    def mbody(j, c):
        acg = acg_v[pl.ds(j * L, L)]
        mrow_v[pl.ds(j * L, L)] = jnp.where(acg > 5.0, 0, 1).astype(jnp.int32)
        return c

    lax.fori_loop(0, B_PER_W // L, mbody, 0)

    cp_g.wait()

    def r2body(i, c):
        @pl.when(mrow_v[pl.ds(i, L)][0] > 0)
        def _():
            _zero_row(rows_v, i)
        return c

    lax.fori_loop(0, B_PER_W, r2body, 0)
    cp_r2 = pltpu.async_copy(rows_v, out_hbm.at[pl.ds(HOT_N + base2, B_PER_W)],
                             sem_r2)

    stores = [None] * len(BS)
    for b in range(len(BS)):
        buf = cbufs[b % NBUF]
        loads[b].wait()

        def cbody(i, c, _s=starts[b], _buf=buf):
            @pl.when(flag_v[pl.ds(_s + i, L)][0] > 0)
            def _():
                _zero_row(_buf, i)
            return c

        lax.fori_loop(0, BS[b], cbody, 0)
        stores[b] = pltpu.async_copy(
            buf.at[pl.ds(0, BS[b])],
            out_hbm.at[pl.ds(HOT_N + B + lo + starts[b], BS[b])],
            ssems[b % NBUF])
        if b + NBUF < len(BS):
            stores[b].wait()
            loads[b + NBUF] = pltpu.async_copy(
                cold_hbm.at[pl.ds(lo + starts[b + NBUF], BS[b + NBUF])],
                cbufs[b % NBUF].at[pl.ds(0, BS[b + NBUF])], lsems[b % NBUF])

    for b in range(max(0, len(BS) - NBUF), len(BS)):
        stores[b].wait()
    cp_r2.wait()
    cp_hs.wait()


def _body_flat(hot_hbm, cold_hbm, ac_hbm, idx_hbm, out_hbm,
               idx_v, acl_v, acg_v, hotbuf, rows_v, flag_v, mrow_v,
               cb0, cb1, cb2, sem_in, sem_hot, sem_g, sem_r2,
               l0, l1, l2, s0, s1, s2):
    _body(hot_hbm, cold_hbm, ac_hbm, idx_hbm, out_hbm,
          idx_v, acl_v, acg_v, hotbuf, rows_v, flag_v, mrow_v,
          (cb0, cb1, cb2), sem_in, sem_hot, sem_g, sem_r2,
          (l0, l1, l2), (s0, s1, s2))


@jax.jit
def kernel(hot_data, cold_data, access_counter, indices):
    kfn = pl.kernel(
        _body_flat,
        out_type=jax.ShapeDtypeStruct((OUT_N, D), jnp.float32),
        scratch_types=[
            pltpu.VMEM((B,), jnp.int32),
            pltpu.VMEM((ACL_W + 2 * L,), jnp.float32),
            pltpu.VMEM((B_PER_W,), jnp.float32),
            pltpu.VMEM((HOT_W, D), jnp.float32),
            pltpu.VMEM((B_PER_W, D), jnp.float32),
            pltpu.VMEM((COLD_W + 2 * L,), jnp.int32),
            pltpu.VMEM((B_PER_W + L,), jnp.int32),
            pltpu.VMEM((BS[0], D), jnp.float32),
            pltpu.VMEM((BS[0], D), jnp.float32),
            pltpu.VMEM((BS[0], D), jnp.float32),
            pltpu.SemaphoreType.DMA,
            pltpu.SemaphoreType.DMA,
            pltpu.SemaphoreType.DMA,
            pltpu.SemaphoreType.DMA,
            pltpu.SemaphoreType.DMA,
            pltpu.SemaphoreType.DMA,
            pltpu.SemaphoreType.DMA,
            pltpu.SemaphoreType.DMA,
            pltpu.SemaphoreType.DMA,
            pltpu.SemaphoreType.DMA,
        ],
        mesh=plsc.VectorSubcoreMesh(core_axis_name="c", subcore_axis_name="s"),
        compiler_params=pltpu.CompilerParams(needs_layout_passes=False,
                                             use_tc_tiling_on_sc=False),
    )
    return kfn(hot_data, cold_data, access_counter, indices)


# speedup vs baseline: 1.7244x; 1.7244x over previous
import jax
import jax.numpy as jnp
from jax import lax
from jax.experimental import pallas as pl
from jax.experimental.pallas import tpu as pltpu
from jax.experimental.pallas import tpu_sc as plsc

HOT_N, COLD_N, D, B = 1000, 10000, 256, 4096
OUT_N = HOT_N + B + COLD_N
NW = 32
L = 16
HOT_W = 32
COLD_S = 312
COLD_W = 328
BS = (80, 80, 80, 88)
NBUF = 3
ACL_W = COLD_W
B_PER_W = B // NW


def _zero_row(buf, i):
    z = jnp.zeros((L,), jnp.float32)
    for col in range(D // L):
        buf[i, pl.ds(col * L, L)] = z


def _body(hot_hbm, cold_hbm, ac_hbm, idx_hbm, out_hbm,
          idx_v, acl_v, acg_v, hotbuf, rows_v, flag_v, mrow_v,
          cbufs, sem_idx, sem_acl, sem_hot, sem_g, sem_ag, sem_r2,
          lsems, ssems):
    wid = lax.axis_index("s") * 2 + lax.axis_index("c")
    lo = wid * COLD_S

    starts = [sum(BS[:b]) for b in range(len(BS))]
    loads = [None] * len(BS)
    for b in range(NBUF):
        loads[b] = pltpu.async_copy(
            cold_hbm.at[pl.ds(lo + starts[b], BS[b])],
            cbufs[b].at[pl.ds(0, BS[b])], lsems[b])
    cp_idx = pltpu.async_copy(idx_hbm, idx_v, sem_idx)
    cp_acl = pltpu.async_copy(ac_hbm.at[pl.ds(lo, ACL_W)],
                              acl_v.at[pl.ds(0, ACL_W)], sem_acl)
    hstart = jnp.minimum(wid * HOT_W, HOT_N - HOT_W)
    cp_hl = pltpu.async_copy(hot_hbm.at[pl.ds(hstart, HOT_W)], hotbuf, sem_hot)

    cp_idx.wait()

    base2 = wid * B_PER_W
    idx2 = idx_v.at[pl.ds(base2, B_PER_W)]
    cp_g = pltpu.async_copy(cold_hbm.at[idx2], rows_v, sem_g)
    cp_ag = pltpu.async_copy(ac_hbm.at[idx2], acg_v, sem_ag)

    cp_hl.wait()
    cp_hs = pltpu.async_copy(hotbuf, out_hbm.at[pl.ds(hstart, HOT_W)], sem_hot)

    def zbody(j, c):
        flag_v[pl.ds(j * L, L)] = jnp.zeros((L,), jnp.int32)
        return c

    lax.fori_loop(0, (COLD_W + 2 * L - 1) // L, zbody, 0)

    ones_i = jnp.ones((L,), jnp.int32)

    def fbody(j, c):
        for u in range(2):
            idx16 = idx_v[pl.ds((2 * j + u) * L, L)]
            m = (idx16 >= lo) & (idx16 < lo + COLD_W)
            loc = jnp.where(m, idx16 - lo, 0)
            plsc.store_scatter(flag_v, [loc], ones_i, mask=m)
        return c

    lax.fori_loop(0, B // (2 * L), fbody, 0)

    cp_acl.wait()

    def combody(j, c):
        hit = flag_v[pl.ds(j * L, L)]
        acb = acl_v[pl.ds(j * L, L)]
        keep = (hit > 0) & (acb > 5.0)
        flag_v[pl.ds(j * L, L)] = jnp.where(keep, 1, 0).astype(jnp.int32)
        return c

    lax.fori_loop(0, (COLD_W + L - 1) // L, combody, 0)

    cp_ag.wait()

    def mbody(j, c):
        acg = acg_v[pl.ds(j * L, L)]
        mrow_v[pl.ds(j * L, L)] = jnp.where(acg > 5.0, 0, 1).astype(jnp.int32)
        return c

    lax.fori_loop(0, B_PER_W // L, mbody, 0)

    cp_g.wait()

    def r2body(i, c):
        @pl.when(mrow_v[pl.ds(i, L)][0] > 0)
        def _():
            _zero_row(rows_v, i)
        return c

    lax.fori_loop(0, B_PER_W, r2body, 0)
    cp_r2 = pltpu.async_copy(rows_v, out_hbm.at[pl.ds(HOT_N + base2, B_PER_W)],
                             sem_r2)

    stores = [None] * len(BS)
    for b in range(len(BS)):
        buf = cbufs[b % NBUF]
        loads[b].wait()

        def cbody(i, c, _s=starts[b], _buf=buf):
            @pl.when(flag_v[pl.ds(_s + i, L)][0] > 0)
            def _():
                _zero_row(_buf, i)
            return c

        lax.fori_loop(0, BS[b], cbody, 0)
        stores[b] = pltpu.async_copy(
            buf.at[pl.ds(0, BS[b])],
            out_hbm.at[pl.ds(HOT_N + B + lo + starts[b], BS[b])],
            ssems[b % NBUF])
        if b + NBUF < len(BS):
            stores[b].wait()
            loads[b + NBUF] = pltpu.async_copy(
                cold_hbm.at[pl.ds(lo + starts[b + NBUF], BS[b + NBUF])],
                cbufs[b % NBUF].at[pl.ds(0, BS[b + NBUF])], lsems[b % NBUF])

    for b in range(max(0, len(BS) - NBUF), len(BS)):
        stores[b].wait()
    cp_r2.wait()
    cp_hs.wait()


def _body_flat(hot_hbm, cold_hbm, ac_hbm, idx_hbm, out_hbm,
               idx_v, acl_v, acg_v, hotbuf, rows_v, flag_v, mrow_v,
               cb0, cb1, cb2, sem_idx, sem_acl, sem_hot, sem_g, sem_ag,
               sem_r2, l0, l1, l2, s0, s1, s2):
    _body(hot_hbm, cold_hbm, ac_hbm, idx_hbm, out_hbm,
          idx_v, acl_v, acg_v, hotbuf, rows_v, flag_v, mrow_v,
          (cb0, cb1, cb2), sem_idx, sem_acl, sem_hot, sem_g, sem_ag,
          sem_r2, (l0, l1, l2), (s0, s1, s2))


@jax.jit
def kernel(hot_data, cold_data, access_counter, indices):
    kfn = pl.kernel(
        _body_flat,
        out_type=jax.ShapeDtypeStruct((OUT_N, D), jnp.float32),
        scratch_types=[
            pltpu.VMEM((B,), jnp.int32),
            pltpu.VMEM((ACL_W + 2 * L,), jnp.float32),
            pltpu.VMEM((B_PER_W,), jnp.float32),
            pltpu.VMEM((HOT_W, D), jnp.float32),
            pltpu.VMEM((B_PER_W, D), jnp.float32),
            pltpu.VMEM((COLD_W + 2 * L,), jnp.int32),
            pltpu.VMEM((B_PER_W + L,), jnp.int32),
            pltpu.VMEM((max(BS), D), jnp.float32),
            pltpu.VMEM((max(BS), D), jnp.float32),
            pltpu.VMEM((max(BS), D), jnp.float32),
            pltpu.SemaphoreType.DMA,
            pltpu.SemaphoreType.DMA,
            pltpu.SemaphoreType.DMA,
            pltpu.SemaphoreType.DMA,
            pltpu.SemaphoreType.DMA,
            pltpu.SemaphoreType.DMA,
            pltpu.SemaphoreType.DMA,
            pltpu.SemaphoreType.DMA,
            pltpu.SemaphoreType.DMA,
            pltpu.SemaphoreType.DMA,
            pltpu.SemaphoreType.DMA,
            pltpu.SemaphoreType.DMA,
        ],
        mesh=plsc.VectorSubcoreMesh(core_axis_name="c", subcore_axis_name="s"),
        compiler_params=pltpu.CompilerParams(needs_layout_passes=False),
    )
    return kfn(hot_data, cold_data, access_counter, indices)


# speedup vs baseline: 1.8131x; 1.0514x over previous
import jax
import jax.numpy as jnp
from jax import lax
from jax.experimental import pallas as pl
from jax.experimental.pallas import tpu as pltpu
from jax.experimental.pallas import tpu_sc as plsc

HOT_N, COLD_N, D, B = 1000, 10000, 256, 4096
OUT_N = HOT_N + B + COLD_N
NW = 32
L = 16
HOT_W = 32
COLD_S = 312
COLD_W = 328
BS = (88, 80, 80, 80)
NBUF = 3
ACL_W = COLD_W
B_PER_W = B // NW


def _zero_row(buf, i):
    z = jnp.zeros((L,), jnp.float32)
    for col in range(D // L):
        buf[i, pl.ds(col * L, L)] = z


def _body(hot_hbm, cold_hbm, ac_hbm, idx_hbm, out_hbm,
          idx_v, acl_v, acg_v, hotbuf, rows_v, flag_v, mrow_v,
          cbufs, sem_idx, sem_acl, sem_hot, sem_g, sem_ag, sem_r2,
          lsems, ssems):
    wid = lax.axis_index("s") * 2 + lax.axis_index("c")
    lo = wid * COLD_S

    cp_idx = pltpu.async_copy(idx_hbm, idx_v, sem_idx)
    cp_acl = pltpu.async_copy(ac_hbm.at[pl.ds(lo, ACL_W)],
                              acl_v.at[pl.ds(0, ACL_W)], sem_acl)
    starts = [sum(BS[:b]) for b in range(len(BS))]
    loads = [None] * len(BS)
    for b in range(NBUF):
        loads[b] = pltpu.async_copy(
            cold_hbm.at[pl.ds(lo + starts[b], BS[b])],
            cbufs[b].at[pl.ds(0, BS[b])], lsems[b])
    hstart = jnp.minimum(wid * HOT_W, HOT_N - HOT_W)
    cp_hl = pltpu.async_copy(hot_hbm.at[pl.ds(hstart, HOT_W)], hotbuf, sem_hot)

    cp_idx.wait()

    base2 = wid * B_PER_W
    idx2 = idx_v.at[pl.ds(base2, B_PER_W)]
    cp_g = pltpu.async_copy(cold_hbm.at[idx2], rows_v, sem_g)
    cp_ag = pltpu.async_copy(ac_hbm.at[idx2], acg_v, sem_ag)

    cp_hl.wait()
    cp_hs = pltpu.async_copy(hotbuf, out_hbm.at[pl.ds(hstart, HOT_W)], sem_hot)

    def zbody(j, c):
        flag_v[pl.ds(j * L, L)] = jnp.zeros((L,), jnp.int32)
        return c

    lax.fori_loop(0, (COLD_W + 2 * L - 1) // L, zbody, 0)

    ones_i = jnp.ones((L,), jnp.int32)

    def fbody(j, c):
        for u in range(2):
            idx16 = idx_v[pl.ds((2 * j + u) * L, L)]
            m = (idx16 >= lo) & (idx16 < lo + COLD_W)
            loc = jnp.where(m, idx16 - lo, 0)
            plsc.store_scatter(flag_v, [loc], ones_i, mask=m)
        return c

    lax.fori_loop(0, B // (2 * L), fbody, 0)

    cp_acl.wait()

    def combody(j, c):
        hit = flag_v[pl.ds(j * L, L)]
        acb = acl_v[pl.ds(j * L, L)]
        keep = (hit > 0) & (acb > 5.0)
        flag_v[pl.ds(j * L, L)] = jnp.where(keep, 1, 0).astype(jnp.int32)
        return c

    lax.fori_loop(0, (COLD_W + L - 1) // L, combody, 0)

    stores = [None] * len(BS)

    def _do_block(b):
        buf = cbufs[b % NBUF]
        loads[b].wait()

        def cbody(i, c, _s=starts[b], _buf=buf):
            @pl.when(flag_v[pl.ds(_s + i, L)][0] > 0)
            def _():
                _zero_row(_buf, i)
            return c

        lax.fori_loop(0, BS[b], cbody, 0)
        stores[b] = pltpu.async_copy(
            buf.at[pl.ds(0, BS[b])],
            out_hbm.at[pl.ds(HOT_N + B + lo + starts[b], BS[b])],
            ssems[b % NBUF])
        if b + NBUF < len(BS):
            stores[b].wait()
            loads[b + NBUF] = pltpu.async_copy(
                cold_hbm.at[pl.ds(lo + starts[b + NBUF], BS[b + NBUF])],
                cbufs[b % NBUF].at[pl.ds(0, BS[b + NBUF])], lsems[b % NBUF])

    _do_block(0)

    cp_ag.wait()

    def mbody(j, c):
        acg = acg_v[pl.ds(j * L, L)]
        mrow_v[pl.ds(j * L, L)] = jnp.where(acg > 5.0, 0, 1).astype(jnp.int32)
        return c

    lax.fori_loop(0, B_PER_W // L, mbody, 0)

    cp_g.wait()

    def r2body(i, c):
        @pl.when(mrow_v[pl.ds(i, L)][0] > 0)
        def _():
            _zero_row(rows_v, i)
        return c

    lax.fori_loop(0, B_PER_W, r2body, 0)
    cp_r2 = pltpu.async_copy(rows_v, out_hbm.at[pl.ds(HOT_N + base2, B_PER_W)],
                             sem_r2)

    for b in range(1, len(BS)):
        _do_block(b)

    for b in range(max(0, len(BS) - NBUF), len(BS)):
        stores[b].wait()
    cp_r2.wait()
    cp_hs.wait()


def _body_flat(hot_hbm, cold_hbm, ac_hbm, idx_hbm, out_hbm,
               idx_v, acl_v, acg_v, hotbuf, rows_v, flag_v, mrow_v,
               cb0, cb1, cb2, sem_idx, sem_acl, sem_hot, sem_g, sem_ag,
               sem_r2, l0, l1, l2, s0, s1, s2):
    _body(hot_hbm, cold_hbm, ac_hbm, idx_hbm, out_hbm,
          idx_v, acl_v, acg_v, hotbuf, rows_v, flag_v, mrow_v,
          (cb0, cb1, cb2), sem_idx, sem_acl, sem_hot, sem_g, sem_ag,
          sem_r2, (l0, l1, l2), (s0, s1, s2))


@jax.jit
def kernel(hot_data, cold_data, access_counter, indices):
    kfn = pl.kernel(
        _body_flat,
        out_type=jax.ShapeDtypeStruct((OUT_N, D), jnp.float32),
        scratch_types=[
            pltpu.VMEM((B,), jnp.int32),
            pltpu.VMEM((ACL_W + 2 * L,), jnp.float32),
            pltpu.VMEM((B_PER_W,), jnp.float32),
            pltpu.VMEM((HOT_W, D), jnp.float32),
            pltpu.VMEM((B_PER_W, D), jnp.float32),
            pltpu.VMEM((COLD_W + 2 * L,), jnp.int32),
            pltpu.VMEM((B_PER_W + L,), jnp.int32),
            pltpu.VMEM((max(BS), D), jnp.float32),
            pltpu.VMEM((max(BS), D), jnp.float32),
            pltpu.VMEM((max(BS), D), jnp.float32),
            pltpu.SemaphoreType.DMA,
            pltpu.SemaphoreType.DMA,
            pltpu.SemaphoreType.DMA,
            pltpu.SemaphoreType.DMA,
            pltpu.SemaphoreType.DMA,
            pltpu.SemaphoreType.DMA,
            pltpu.SemaphoreType.DMA,
            pltpu.SemaphoreType.DMA,
            pltpu.SemaphoreType.DMA,
            pltpu.SemaphoreType.DMA,
            pltpu.SemaphoreType.DMA,
            pltpu.SemaphoreType.DMA,
        ],
        mesh=plsc.VectorSubcoreMesh(core_axis_name="c", subcore_axis_name="s"),
        compiler_params=pltpu.CompilerParams(needs_layout_passes=False),
    )
    return kfn(hot_data, cold_data, access_counter, indices)
